# baseline (speedup 1.0000x reference)
import functools
import math

import jax
import jax.numpy as jnp
from jax import lax
from jax.experimental import pallas as pl
from jax.experimental.pallas import tpu as pltpu
from jax.experimental.pallas import tpu_sc as plsc

MIN_NORM = 1e-15
MAXNORM = 1.0 - 4e-3
ATH_MAX = 0.5 * math.log((1.0 + MAXNORM) / (1.0 - MAXNORM))
WROW = 48
DCOL = 32
CH = 128
BCH = 512
NW = 32


def _artanh(z):
    z = jnp.clip(z, -1.0 + 1e-7, 1.0 - 1e-7)
    return 0.5 * jnp.log((1.0 + z) / (1.0 - z))


def _chunk_masks():
    c = lax.broadcasted_iota(jnp.int32, (4 * WROW, 4), 0)
    kk = lax.broadcasted_iota(jnp.int32, (4 * WROW, 4), 1)
    mc = (c // WROW == kk).astype(jnp.float32)
    cb = lax.broadcasted_iota(jnp.int32, (4, 4 * WROW), 1)
    kb = lax.broadcasted_iota(jnp.int32, (4, 4 * WROW), 0)
    bc = (cb // WROW == kb).astype(jnp.float32)
    return mc, bc


def _mm(a, b):
    return jnp.dot(a, b, preferred_element_type=jnp.float32)


def _stage1_body(x_ref, w_ref, b_ref, o0_ref, o1_ref, o2_ref, o3_ref):
    mc, bc = _chunk_masks()
    x = x_ref[...]
    onesf = jnp.ones((x.shape[1], 1), jnp.float32)
    xn = jnp.maximum(jnp.sqrt(_mm(x * x, onesf)), MIN_NORM)
    th = jnp.maximum(jnp.tanh(xn), MIN_NORM)
    xh = x * (jnp.minimum(th, MAXNORM) / xn)
    xnh = jnp.maximum(jnp.minimum(th, MAXNORM), MIN_NORM)
    rat = jnp.minimum(xn, ATH_MAX) / xnh
    mx = jnp.dot(xh, w_ref[...], preferred_element_type=jnp.float32)
    mq4 = _mm(mx * mx, mc)
    mxn4 = jnp.maximum(jnp.sqrt(mq4), MIN_NORM)
    g4 = jnp.tanh(mxn4 * rat)
    gc4 = jnp.minimum(g4, MAXNORM)
    nz = mq4 > 0.0
    s4 = jnp.where(nz, gc4 / mxn4, 0.0)
    x2_4 = jnp.where(nz, gc4 * gc4, 0.0)
    h = mx * _mm(s4, bc)
    b = b_ref[...]
    bn4 = jnp.maximum(jnp.sqrt(_mm(b * b, mc)), MIN_NORM)
    hb = b * _mm(jnp.minimum(jnp.maximum(jnp.tanh(bn4), MIN_NORM), MAXNORM) / bn4, bc)
    y2_4 = _mm(hb * hb, mc)
    xy4 = _mm(h * hb, mc)
    al4 = 1.0 + 2.0 * xy4 + y2_4
    be4 = 1.0 - x2_4
    den4 = 1.0 + 2.0 * xy4 + x2_4 * y2_4
    ha = (h * _mm(al4, bc) + hb * _mm(be4, bc)) * _mm(
        1.0 / jnp.maximum(den4, MIN_NORM), bc)
    an2_4 = _mm(ha * ha, mc)
    an4 = jnp.maximum(jnp.sqrt(an2_4), MIN_NORM)
    sc4 = _artanh(jnp.minimum(an4, MAXNORM)) / an4
    lane = lax.broadcasted_iota(jnp.int32, (1, 4 * WROW), 1)
    onecol = (lane % WROW == DCOL).astype(jnp.float32)
    res = ha * _mm(sc4, bc) + onecol
    for k, o_ref in enumerate((o0_ref, o1_ref, o2_ref, o3_ref)):
        o_ref[...] = res[:, k * WROW : (k + 1) * WROW]


def _stage3_body(t0_ref, t1_ref, t2_ref, t3_ref,
                 p0_ref, p1_ref, p2_ref, p3_ref, out_ref):
    lane = lax.broadcasted_iota(jnp.int32, (1, WROW), 1)
    fmask = (lane < DCOL).astype(jnp.float32)
    degsel = (lax.broadcasted_iota(jnp.int32, (WROW, 1), 0) == DCOL
              ).astype(jnp.float32)
    ones48 = jnp.ones((WROW, 1), jnp.float32)
    tws, lgs, w1s = [], [], []
    for p_ref, t_ref in ((p0_ref, t0_ref), (p1_ref, t1_ref),
                         (p2_ref, t2_ref), (p3_ref, t3_ref)):
        aggf = p_ref[0] + p_ref[1] + t_ref[...]
        deg1 = _mm(aggf, degsel)
        m = aggf * fmask
        s2 = _mm(m * m, ones48)
        s = jnp.maximum(jnp.sqrt(s2) / deg1, MIN_NORM)
        cs = jnp.minimum(s, ATH_MAX) / (s * deg1)
        xt = jnp.maximum(m, 0.0) * cs
        t2 = _mm(xt * xt, ones48)
        t = jnp.maximum(jnp.sqrt(t2), MIN_NORM)
        tc = jnp.minimum(t, ATH_MAX)
        invt = 1.0 / t
        w1 = jnp.tanh(0.125 * tc)
        w1s.append(w1)
        tws.append(xt * (w1 * invt))
        lgs.append(xt * (tc * invt))
    target = tws[0]
    for k in range(1, 4):
        b = tws[k]
        b2 = w1s[k] * w1s[k]
        a2 = _mm(target * target, ones48)
        ab = _mm(target * b, ones48)
        num = (1.0 + 2.0 * ab + b2) * target + (1.0 - a2) * b
        den = 1.0 + 2.0 * ab + a2 * b2
        target = num * (1.0 / jnp.maximum(den, MIN_NORM))
    tn = jnp.maximum(jnp.sqrt(_mm(target * target, ones48)), MIN_NORM)
    acc = lgs[0] + lgs[1] + lgs[2] + lgs[3] + target * (_artanh(tn) / tn)
    nr = jnp.maximum(0.2 * jnp.sqrt(_mm(acc * acc, ones48)), MIN_NORM)
    out = acc * (0.2 * jnp.minimum(jnp.maximum(jnp.tanh(nr), MIN_NORM), MAXNORM) / nr)
    out_ref[...] = out[:, :DCOL]


def _make_sc_agg(n_pad, stripe, rows_total):
    mesh = plsc.VectorSubcoreMesh(core_axis_name="c", subcore_axis_name="s")
    nbuf = 2
    wrows = rows_total // NW
    ntri = -(-wrows // nbuf)

    @functools.partial(
        pl.kernel,
        mesh=mesh,
        compiler_params=pltpu.CompilerParams(use_tc_tiling_on_sc=False),
        out_type=jax.ShapeDtypeStruct((2, n_pad, WROW), jnp.float32),
        scratch_types=[
            pltpu.VMEM((CH, WROW), jnp.float32),
            pltpu.VMEM((stripe, WROW), jnp.float32),
            pltpu.VMEM((wrows, BCH), jnp.int32),
            pltpu.VMEM((wrows, BCH), jnp.int32),
            pltpu.VMEM((nbuf, BCH, WROW), jnp.float32),
            pltpu.VMEM_SHARED((n_pad, WROW), jnp.float32),
            [pltpu.SemaphoreType.DMA] * nbuf,
            [pltpu.SemaphoreType.DMA] * nbuf,
        ],
    )
    def sc_agg(tab_k, e_hbm, out_hbm,
               zbuf, stage, src_v, dst_v, rows, acc, gsem, ssem):
        cid = lax.axis_index("c")
        sid = lax.axis_index("s")
        wid = cid * 16 + sid
        lo = wid * wrows
        zero16 = jnp.zeros((16,), jnp.float32)

        def zrow(i, carry):
            for q in range(WROW // 16):
                zbuf[i, pl.ds(q * 16, 16)] = zero16
            return carry

        lax.fori_loop(0, CH, zrow, 0)

        for t in range(stripe // CH):
            pltpu.sync_copy(zbuf, acc.at[pl.ds(sid * stripe + t * CH, CH)])
        pltpu.sync_copy(e_hbm.at[0, pl.ds(lo, wrows)], src_v)
        pltpu.sync_copy(e_hbm.at[1, pl.ds(lo, wrows)], dst_v)
        plsc.subcore_barrier()

        def gath(q, j):
            pltpu.async_copy(
                tab_k.at[src_v.at[j]], rows.at[q], gsem[q])

        for q in range(nbuf):
            @pl.when(q < wrows)
            def _(q=q):
                gath(q, q)

        def tri(i, carry):
            c0 = nbuf * i
            for q in range(nbuf):
                @pl.when(c0 + q < wrows)
                def _(q=q):
                    j = c0 + q
                    pltpu.make_async_copy(
                        tab_k.at[src_v.at[j]], rows.at[q], gsem[q]).wait()
                    pltpu.async_copy(
                        rows.at[q], acc.at[dst_v.at[j]], ssem[q], add=True)
            for q in range(nbuf):
                @pl.when(c0 + q + nbuf < wrows)
                def _(q=q):
                    j = c0 + q
                    pltpu.make_async_copy(
                        rows.at[q], acc.at[dst_v.at[j]], ssem[q]).wait()
                    gath(q, j + nbuf)
            return carry

        lax.fori_loop(0, ntri, tri, 0)
        for q in range(nbuf):
            @pl.when(q < wrows)
            def _(q=q):
                pltpu.make_async_copy(
                    rows.at[q], acc.at[dst_v.at[0]], ssem[q]).wait()
        plsc.subcore_barrier()

        pltpu.sync_copy(acc.at[pl.ds(sid * stripe, stripe)], stage)
        pltpu.sync_copy(stage, out_hbm.at[cid, pl.ds(sid * stripe, stripe)])

    return sc_agg


def kernel(x, adj, k_diffusion_in, k_diffusion_out, k_neighbor_in, k_neighbor_out,
           W_di, b_di, W_do, b_do, W_ni, b_ni, W_no, b_no):
    del adj
    n, f = x.shape
    e = k_diffusion_in.shape[-1]

    ws = [W_di, W_do, W_ni, W_no]
    bs = [b_di, b_do, b_ni, b_no]
    d = ws[0].shape[0]
    w_pack = jnp.concatenate(
        [jnp.pad(w.T, ((0, 0), (0, WROW - d))) for w in ws], axis=1)
    b_pack = jnp.concatenate(
        [jnp.pad(b, (0, WROW - d)) for b in bs]).reshape(1, 4 * WROW)

    stripe = 640
    n_pad = 16 * stripe

    ep = -(-e // (NW * BCH)) * (NW * BCH)

    def prep(edges):
        ei = edges[0]
        if ep != e:
            pad_dst = n + (jnp.arange(ep - e, dtype=ei.dtype) % (n_pad - n))
            src = jnp.concatenate([ei[0], jnp.zeros((ep - e,), ei.dtype)])
            dst = jnp.concatenate([ei[1], pad_dst])
            ei = jnp.stack([src, dst])
        return ei.reshape(2, -1, BCH)

    edge_views = [prep(t) for t in (k_diffusion_in, k_diffusion_out,
                                    k_neighbor_in, k_neighbor_out)]
    rows_total = edge_views[0].shape[1]

    r = 1000
    tspec = pl.BlockSpec((r, WROW), lambda i: (i, 0))
    tables = pl.pallas_call(
        _stage1_body,
        grid=(n // r,),
        in_specs=[
            pl.BlockSpec((r, f), lambda i: (i, 0)),
            pl.BlockSpec((f, 4 * WROW), lambda i: (0, 0)),
            pl.BlockSpec((1, 4 * WROW), lambda i: (0, 0)),
        ],
        out_specs=[tspec] * 4,
        out_shape=[jax.ShapeDtypeStruct((n, WROW), jnp.float32)] * 4,
    )(x, w_pack, b_pack)

    sc_call = _make_sc_agg(n_pad, stripe, rows_total)
    partials = [sc_call(tables[k], edge_views[k]) for k in range(4)]

    pspec = pl.BlockSpec((2, r, WROW), lambda i: (0, i, 0))
    out = pl.pallas_call(
        _stage3_body,
        grid=(n // r,),
        in_specs=[tspec] * 4 + [pspec] * 4,
        out_specs=pl.BlockSpec((r, DCOL), lambda i: (i, 0)),
        out_shape=jax.ShapeDtypeStruct((n, DCOL), jnp.float32),
    )(*tables, *partials)
    return out


# speedup vs baseline: 2.2085x; 2.2085x over previous
import functools
import math

import jax
import jax.numpy as jnp
from jax import lax
from jax.experimental import pallas as pl
from jax.experimental.pallas import tpu as pltpu
from jax.experimental.pallas import tpu_sc as plsc

MIN_NORM = 1e-15
MAXNORM = 1.0 - 4e-3
ATH_MAX = 0.5 * math.log((1.0 + MAXNORM) / (1.0 - MAXNORM))
WROW = 48
DCOL = 32
CH = 128
NW = 32


def _artanh(z):
    z = jnp.clip(z, -1.0 + 1e-7, 1.0 - 1e-7)
    return 0.5 * jnp.log((1.0 + z) / (1.0 - z))


def _chunk_masks():
    c = lax.broadcasted_iota(jnp.int32, (4 * WROW, 4), 0)
    kk = lax.broadcasted_iota(jnp.int32, (4 * WROW, 4), 1)
    mc = (c // WROW == kk).astype(jnp.float32)
    cb = lax.broadcasted_iota(jnp.int32, (4, 4 * WROW), 1)
    kb = lax.broadcasted_iota(jnp.int32, (4, 4 * WROW), 0)
    bc = (cb // WROW == kb).astype(jnp.float32)
    return mc, bc


def _mm(a, b):
    return jnp.dot(a, b, preferred_element_type=jnp.float32)


def _stage1_body(x_ref, w_ref, b_ref, o0_ref, o1_ref, o2_ref, o3_ref):
    mc, bc = _chunk_masks()
    x = x_ref[...]
    onesf = jnp.ones((x.shape[1], 1), jnp.float32)
    xn = jnp.maximum(jnp.sqrt(_mm(x * x, onesf)), MIN_NORM)
    th = jnp.maximum(jnp.tanh(xn), MIN_NORM)
    xh = x * (jnp.minimum(th, MAXNORM) / xn)
    xnh = jnp.maximum(jnp.minimum(th, MAXNORM), MIN_NORM)
    rat = jnp.minimum(xn, ATH_MAX) / xnh
    mx = jnp.dot(xh, w_ref[...], preferred_element_type=jnp.float32)
    mq4 = _mm(mx * mx, mc)
    mxn4 = jnp.maximum(jnp.sqrt(mq4), MIN_NORM)
    g4 = jnp.tanh(mxn4 * rat)
    gc4 = jnp.minimum(g4, MAXNORM)
    nz = mq4 > 0.0
    s4 = jnp.where(nz, gc4 / mxn4, 0.0)
    x2_4 = jnp.where(nz, gc4 * gc4, 0.0)
    h = mx * _mm(s4, bc)
    b = b_ref[...]
    bn4 = jnp.maximum(jnp.sqrt(_mm(b * b, mc)), MIN_NORM)
    hb = b * _mm(jnp.minimum(jnp.maximum(jnp.tanh(bn4), MIN_NORM), MAXNORM) / bn4, bc)
    y2_4 = _mm(hb * hb, mc)
    xy4 = _mm(h * hb, mc)
    al4 = 1.0 + 2.0 * xy4 + y2_4
    be4 = 1.0 - x2_4
    den4 = 1.0 + 2.0 * xy4 + x2_4 * y2_4
    ha = (h * _mm(al4, bc) + hb * _mm(be4, bc)) * _mm(
        1.0 / jnp.maximum(den4, MIN_NORM), bc)
    an2_4 = _mm(ha * ha, mc)
    an4 = jnp.maximum(jnp.sqrt(an2_4), MIN_NORM)
    sc4 = _artanh(jnp.minimum(an4, MAXNORM)) / an4
    lane = lax.broadcasted_iota(jnp.int32, (1, 4 * WROW), 1)
    onecol = (lane % WROW == DCOL).astype(jnp.float32)
    res = ha * _mm(sc4, bc) + onecol
    for k, o_ref in enumerate((o0_ref, o1_ref, o2_ref, o3_ref)):
        o_ref[...] = res[:, k * WROW : (k + 1) * WROW]


def _stage3_body(t0_ref, t1_ref, t2_ref, t3_ref,
                 p0_ref, p1_ref, p2_ref, p3_ref, out_ref):
    lane = lax.broadcasted_iota(jnp.int32, (1, WROW), 1)
    fmask = (lane < DCOL).astype(jnp.float32)
    degsel = (lax.broadcasted_iota(jnp.int32, (WROW, 1), 0) == DCOL
              ).astype(jnp.float32)
    ones48 = jnp.ones((WROW, 1), jnp.float32)
    tws, lgs, w1s = [], [], []
    for p_ref, t_ref in ((p0_ref, t0_ref), (p1_ref, t1_ref),
                         (p2_ref, t2_ref), (p3_ref, t3_ref)):
        aggf = p_ref[0] + p_ref[1] + t_ref[...]
        deg1 = _mm(aggf, degsel)
        m = aggf * fmask
        s2 = _mm(m * m, ones48)
        s = jnp.maximum(jnp.sqrt(s2) / deg1, MIN_NORM)
        cs = jnp.minimum(s, ATH_MAX) / (s * deg1)
        xt = jnp.maximum(m, 0.0) * cs
        t2 = _mm(xt * xt, ones48)
        t = jnp.maximum(jnp.sqrt(t2), MIN_NORM)
        tc = jnp.minimum(t, ATH_MAX)
        invt = 1.0 / t
        w1 = jnp.tanh(0.125 * tc)
        w1s.append(w1)
        tws.append(xt * (w1 * invt))
        lgs.append(xt * (tc * invt))
    target = tws[0]
    for k in range(1, 4):
        b = tws[k]
        b2 = w1s[k] * w1s[k]
        a2 = _mm(target * target, ones48)
        ab = _mm(target * b, ones48)
        num = (1.0 + 2.0 * ab + b2) * target + (1.0 - a2) * b
        den = 1.0 + 2.0 * ab + a2 * b2
        target = num * (1.0 / jnp.maximum(den, MIN_NORM))
    tn = jnp.maximum(jnp.sqrt(_mm(target * target, ones48)), MIN_NORM)
    acc = lgs[0] + lgs[1] + lgs[2] + lgs[3] + target * (_artanh(tn) / tn)
    nr = jnp.maximum(0.2 * jnp.sqrt(_mm(acc * acc, ones48)), MIN_NORM)
    out = acc * (0.2 * jnp.minimum(jnp.maximum(jnp.tanh(nr), MIN_NORM), MAXNORM) / nr)
    out_ref[...] = out[:, :DCOL]


def _make_sc_agg(n_pad, stripe, rows_total, wrows_max):
    mesh = plsc.VectorSubcoreMesh(core_axis_name="c", subcore_axis_name="s")
    nquad = -(-wrows_max // 4)

    @functools.partial(
        pl.kernel,
        mesh=mesh,
        compiler_params=pltpu.CompilerParams(use_tc_tiling_on_sc=False),
        out_type=jax.ShapeDtypeStruct((2, n_pad, WROW), jnp.float32),
        scratch_types=[
            pltpu.VMEM((CH, WROW), jnp.float32),
            pltpu.VMEM((stripe, WROW), jnp.float32),
            pltpu.VMEM((wrows_max, CH), jnp.int32),
            pltpu.VMEM((wrows_max, CH), jnp.int32),
            pltpu.VMEM((4, CH, WROW), jnp.float32),
            pltpu.VMEM_SHARED((n_pad, WROW), jnp.float32),
            [pltpu.SemaphoreType.DMA] * 4,
            [pltpu.SemaphoreType.DMA] * 4,
        ],
    )
    def sc_agg(tab_k, e_hbm, out_hbm,
               zbuf, stage, src_v, dst_v, rows, acc, gsem, ssem):
        cid = lax.axis_index("c")
        sid = lax.axis_index("s")
        wid = cid * 16 + sid
        lo = wid * rows_total // NW
        cnt = (wid + 1) * rows_total // NW - lo
        zero16 = jnp.zeros((16,), jnp.float32)

        def zrow(i, carry):
            for q in range(WROW // 16):
                zbuf[i, pl.ds(q * 16, 16)] = zero16
            return carry

        lax.fori_loop(0, CH, zrow, 0)

        for t in range(stripe // CH):
            pltpu.sync_copy(zbuf, acc.at[pl.ds(sid * stripe + t * CH, CH)])
        pltpu.sync_copy(e_hbm.at[0, pl.ds(lo, wrows_max)], src_v)
        pltpu.sync_copy(e_hbm.at[1, pl.ds(lo, wrows_max)], dst_v)
        plsc.subcore_barrier()

        for b in range(4):
            @pl.when(b < cnt)
            def _(b=b):
                pltpu.async_copy(tab_k.at[src_v.at[b]], rows.at[b], gsem[b])

        def quad(i, carry):
            j0 = 4 * i
            for b in range(4):
                @pl.when(j0 + b < cnt)
                def _(b=b):
                    j = j0 + b
                    pltpu.make_async_copy(
                        tab_k.at[src_v.at[j]], rows.at[b], gsem[b]).wait()
                    pltpu.async_copy(
                        rows.at[b], acc.at[dst_v.at[j]], ssem[b], add=True)
            for b in range(4):
                @pl.when(j0 + b + 4 < cnt)
                def _(b=b):
                    j = j0 + b
                    pltpu.make_async_copy(
                        rows.at[b], acc.at[dst_v.at[j]], ssem[b]).wait()
                    pltpu.async_copy(
                        tab_k.at[src_v.at[j + 4]], rows.at[b], gsem[b])
            return carry

        lax.fori_loop(0, nquad, quad, 0)
        for b in range(4):
            @pl.when(b < cnt)
            def _(b=b):
                pltpu.make_async_copy(
                    rows.at[b], acc.at[dst_v.at[0]], ssem[b]).wait()
        plsc.subcore_barrier()

        pltpu.sync_copy(acc.at[pl.ds(sid * stripe, stripe)], stage)
        pltpu.sync_copy(stage, out_hbm.at[cid, pl.ds(sid * stripe, stripe)])

    return sc_agg


def kernel(x, adj, k_diffusion_in, k_diffusion_out, k_neighbor_in, k_neighbor_out,
           W_di, b_di, W_do, b_do, W_ni, b_ni, W_no, b_no):
    del adj
    n, f = x.shape
    e = k_diffusion_in.shape[-1]

    ws = [W_di, W_do, W_ni, W_no]
    bs = [b_di, b_do, b_ni, b_no]
    d = ws[0].shape[0]
    w_pack = jnp.concatenate(
        [jnp.pad(w.T, ((0, 0), (0, WROW - d))) for w in ws], axis=1)
    b_pack = jnp.concatenate(
        [jnp.pad(b, (0, WROW - d)) for b in bs]).reshape(1, 4 * WROW)

    stripe = 640
    n_pad = 16 * stripe

    def prep(edges):
        ei = edges[0]
        if e % CH:
            ep = -(-e // CH) * CH
            pad_dst = n + (jnp.arange(ep - e, dtype=ei.dtype) % (n_pad - n))
            src = jnp.concatenate([ei[0], jnp.zeros((ep - e,), ei.dtype)])
            dst = jnp.concatenate([ei[1], pad_dst])
            ei = jnp.stack([src, dst])
        return ei.reshape(2, -1, CH)

    edge_views = [prep(t) for t in (k_diffusion_in, k_diffusion_out,
                                    k_neighbor_in, k_neighbor_out)]
    rows_total = edge_views[0].shape[1]
    wrows_max = -(-rows_total // NW)

    r = 1000
    tspec = pl.BlockSpec((r, WROW), lambda i: (i, 0))
    tables = pl.pallas_call(
        _stage1_body,
        grid=(n // r,),
        in_specs=[
            pl.BlockSpec((r, f), lambda i: (i, 0)),
            pl.BlockSpec((f, 4 * WROW), lambda i: (0, 0)),
            pl.BlockSpec((1, 4 * WROW), lambda i: (0, 0)),
        ],
        out_specs=[tspec] * 4,
        out_shape=[jax.ShapeDtypeStruct((n, WROW), jnp.float32)] * 4,
    )(x, w_pack, b_pack)

    sc_call = _make_sc_agg(n_pad, stripe, rows_total, wrows_max)
    partials = [sc_call(tables[k], edge_views[k]) for k in range(4)]

    pspec = pl.BlockSpec((2, r, WROW), lambda i: (0, i, 0))
    out = pl.pallas_call(
        _stage3_body,
        grid=(n // r,),
        in_specs=[tspec] * 4 + [pspec] * 4,
        out_specs=pl.BlockSpec((r, DCOL), lambda i: (i, 0)),
        out_shape=jax.ShapeDtypeStruct((n, DCOL), jnp.float32),
    )(*tables, *partials)
    return out
